# Initial kernel scaffold; baseline (speedup 1.0000x reference)
import jax
import jax.numpy as jnp
from jax.experimental import pallas as pl


def kernel(state):
    raise NotImplementedError("write your pallas kernel here")


# speedup vs baseline: 867.2139x; 867.2139x over previous
import functools

import jax
import jax.numpy as jnp
from jax.experimental import pallas as pl

_N_T, _N_Y, _N_X = 10, 256, 256
_KAPPA, _TAU, _DT = 0.33, 1.0, 1.0
_C = 1.0 / (_TAU ** 2 * _DT)

_OFFS = ((0, 0), (0, 1), (0, -1), (1, 0), (-1, 0), (1, 1), (-1, -1), (1, -1), (-1, 1))


def _cyc(v, oy, ox):
    if oy == 1:
        v = jnp.concatenate([v[1:, :], v[:1, :]], axis=0)
    elif oy == -1:
        v = jnp.concatenate([v[-1:, :], v[:-1, :]], axis=0)
    if ox == 1:
        v = jnp.concatenate([v[:, 1:], v[:, :1]], axis=1)
    elif ox == -1:
        v = jnp.concatenate([v[:, -1:], v[:, :-1]], axis=1)
    return v


def _phi_kernel(state_ref, out_ref):
    k = pl.program_id(1)

    vx = state_ref[0, _N_T]
    vy = state_ref[0, _N_T + 1]

    iy = jax.lax.broadcasted_iota(jnp.int32, (_N_Y, _N_X), 0)
    ix = jax.lax.broadcasted_iota(jnp.int32, (_N_Y, _N_X), 1)
    hi = _N_X - 1

    hxx = 1.0 + vx * vx
    hyy = 1.0 + vy * vy
    wd = 0.5 * (vx * vy)
    wself = _KAPPA ** 2 + 2.0 * hxx + 2.0 * hyy

    mxp = jnp.where(ix < hi, 1.0, 0.0).astype(jnp.float32)
    mxm = jnp.where(ix > 0, 1.0, 0.0).astype(jnp.float32)
    myp = jnp.where(iy < hi, 1.0, 0.0).astype(jnp.float32)
    mym = jnp.where(iy > 0, 1.0, 0.0).astype(jnp.float32)
    W = {
        (0, 0): wself,
        (0, 1): -hxx * mxp,
        (0, -1): -hxx * mxm,
        (1, 0): -hyy * myp,
        (-1, 0): -hyy * mym,
        (1, 1): -wd * (myp * mxp),
        (-1, -1): -wd * (mym * mxm),
        (1, -1): wd * (myp * mxm),
        (-1, 1): wd * (mym * mxp),
    }

    def A(v):
        acc = W[(0, 0)] * v
        for o in _OFFS[1:]:
            acc = acc + W[o] * _cyc(v, o[0], o[1])
        return acc

    def At(v):
        acc = W[(0, 0)] * v
        for o in _OFFS[1:]:
            acc = acc + _cyc(W[o] * v, -o[0], -o[1])
        return acc

    km = jnp.maximum(k - 1, 0)
    kp = jnp.minimum(k + 1, _N_T - 1)
    xm = state_ref[0, km]
    xc = state_ref[0, k]
    xp = state_ref[0, kp]

    @pl.when(k == 0)
    def _():
        a0 = A(xc)
        at0 = At(xc)
        q0 = 0.5 * (At(a0) + A(at0)) + 0.05 * xc
        out_ref[0, 0] = q0 + xc - _C * (xp + A(xp))

    @pl.when((k > 0) & (k < _N_T - 1))
    def _():
        u = xc + A(xc)
        z = u + At(u)
        w = xm + At(xm)
        up = xp + A(xp)
        out_ref[0, 0] = _C * (z + xc - w - up)

    @pl.when(k == _N_T - 1)
    def _():
        u = xc + A(xc)
        z = u + At(u)
        w = xm + At(xm)
        out_ref[0, 0] = _C * (z - w)


@jax.jit
def kernel(state):
    nb = state.shape[0]
    return pl.pallas_call(
        _phi_kernel,
        grid=(nb, _N_T),
        in_specs=[pl.BlockSpec((1, _N_T + 2, _N_Y, _N_X), lambda b, k: (b, 0, 0, 0))],
        out_specs=pl.BlockSpec((1, 1, _N_Y, _N_X), lambda b, k: (b, k, 0, 0)),
        out_shape=jax.ShapeDtypeStruct((nb, _N_T, _N_Y, _N_X), state.dtype),
    )(state)
